# baseline (speedup 1.0000x reference)
import functools

import jax
import jax.numpy as jnp
from jax.experimental import pallas as pl
from jax.experimental.pallas import tpu as pltpu

_HIDDEN = 768
_INTER = 2048
_BM = 128


def _gemm_body(be_ref, nact_ref, x_ref, wg_ref, wu_ref, down_ref, pw_ref,
               o_ref):
    g = pl.program_id(0)

    @pl.when(g < nact_ref[0])
    def _():
        x = x_ref[...]
        gate = jax.lax.dot_general(
            x, wg_ref[0], (((1,), (1,)), ((), ())),
            preferred_element_type=jnp.float32)
        up = jax.lax.dot_general(
            x, wu_ref[0], (((1,), (1,)), ((), ())),
            preferred_element_type=jnp.float32)
        act = gate * jax.nn.sigmoid(gate) * up
        dn = down_ref[0]
        o = jax.lax.dot_general(
            act, dn, (((1,), (1,)), ((), ())),
            preferred_element_type=jnp.float32)
        o_ref[...] = o * pw_ref[...]


def _grouped_gemm(x_padded, w13, down_proj, pw_padded, block_expert,
                  num_active, interpret=False):
    G = block_expert.shape[0]
    H, I = _HIDDEN, _INTER
    grid_spec = pltpu.PrefetchScalarGridSpec(
        num_scalar_prefetch=2,
        grid=(G,),
        in_specs=[
            pl.BlockSpec((_BM, H), lambda g, be, na: (g, 0)),
            pl.BlockSpec((1, I, H), lambda g, be, na: (be[g], 0, 0)),
            pl.BlockSpec((1, I, H), lambda g, be, na: (be[g], 1, 0)),
            pl.BlockSpec((1, H, I), lambda g, be, na: (be[g], 0, 0)),
            pl.BlockSpec((_BM, 1), lambda g, be, na: (g, 0)),
        ],
        out_specs=pl.BlockSpec((_BM, H), lambda g, be, na: (g, 0)),
    )
    return pl.pallas_call(
        _gemm_body,
        grid_spec=grid_spec,
        out_shape=jax.ShapeDtypeStruct((G * _BM, H), jnp.float32),
        interpret=interpret,
    )(block_expert, num_active, x_padded, w13, w13, down_proj, pw_padded)


@functools.partial(jax.jit, static_argnames=("interpret",))
def _moe(x, expert_indices, expert_weights, w13, down_proj, interpret=False):
    B, S, H = x.shape
    E = w13.shape[0]
    top_k = expert_indices.shape[-1]
    N = B * S
    P = N * top_k
    G = P // _BM + E
    P_pad = G * _BM

    x_flat = x.reshape(N, H)
    flat_e = expert_indices.reshape(P)
    w_flat = expert_weights.reshape(P)

    order = jnp.argsort(flat_e)
    e_sorted = flat_e[order]
    sizes = jnp.bincount(flat_e, length=E)
    blocks_e = (sizes + _BM - 1) // _BM
    starts_unpad = jnp.cumsum(sizes) - sizes
    block_start_e = jnp.cumsum(blocks_e) - blocks_e
    starts_pad = block_start_e * _BM
    ranks = jnp.arange(P, dtype=jnp.int32) - starts_unpad[e_sorted]
    pos_sorted = (starts_pad[e_sorted] + ranks).astype(jnp.int32)

    tok_padded = jnp.zeros((P_pad,), jnp.int32).at[pos_sorted].set(
        (order // top_k).astype(jnp.int32))
    pw_padded = jnp.zeros((P_pad, 1), jnp.float32).at[pos_sorted, 0].set(
        w_flat[order])
    pos_by_pair = jnp.zeros((P,), jnp.int32).at[order].set(pos_sorted)

    num_active = jnp.sum(blocks_e).astype(jnp.int32).reshape(1)
    block_expert = jnp.minimum(
        jnp.searchsorted(jnp.cumsum(blocks_e), jnp.arange(G), side="right"),
        E - 1).astype(jnp.int32)

    x_padded = x_flat[tok_padded]

    o_padded = _grouped_gemm(x_padded, w13, down_proj, pw_padded,
                             block_expert, num_active, interpret=interpret)

    pos2 = pos_by_pair.reshape(N, top_k)
    out = jnp.sum(o_padded[pos2], axis=1)
    return out.reshape(B, S, H)


def kernel(x, expert_indices, expert_weights, w13, down_proj):
    return _moe(x, expert_indices, expert_weights, w13, down_proj)


# speedup vs baseline: 1.0868x; 1.0868x over previous
import functools

import jax
import jax.numpy as jnp
from jax.experimental import pallas as pl
from jax.experimental.pallas import tpu as pltpu

_HIDDEN = 768
_INTER = 2048
_BM = 128


def _gemm_body(be_ref, nact_ref, x_ref, wg_ref, wu_ref, down_ref, pw_ref,
               o_ref):
    g = pl.program_id(0)

    @pl.when(g < nact_ref[0])
    def _():
        x = x_ref[...]
        gate = jax.lax.dot_general(
            x, wg_ref[0], (((1,), (1,)), ((), ())),
            preferred_element_type=jnp.float32)
        up = jax.lax.dot_general(
            x, wu_ref[0], (((1,), (1,)), ((), ())),
            preferred_element_type=jnp.float32)
        act = gate * jax.nn.sigmoid(gate) * up
        dn = down_ref[0]
        o = jax.lax.dot_general(
            act, dn, (((1,), (1,)), ((), ())),
            preferred_element_type=jnp.float32)
        o_ref[...] = o * pw_ref[...]


def _grouped_gemm(x_padded, w13, down_proj, pw_padded, block_expert,
                  num_active, interpret=False):
    G = block_expert.shape[0]
    H, I = _HIDDEN, _INTER
    grid_spec = pltpu.PrefetchScalarGridSpec(
        num_scalar_prefetch=2,
        grid=(G,),
        in_specs=[
            pl.BlockSpec((_BM, H), lambda g, be, na: (g, 0)),
            pl.BlockSpec((1, I, H), lambda g, be, na: (be[g], 0, 0)),
            pl.BlockSpec((1, I, H), lambda g, be, na: (be[g], 1, 0)),
            pl.BlockSpec((1, H, I), lambda g, be, na: (be[g], 0, 0)),
            pl.BlockSpec((_BM, 1), lambda g, be, na: (g, 0)),
        ],
        out_specs=pl.BlockSpec((_BM, H), lambda g, be, na: (g, 0)),
    )
    return pl.pallas_call(
        _gemm_body,
        grid_spec=grid_spec,
        out_shape=jax.ShapeDtypeStruct((G * _BM, H), jnp.float32),
        interpret=interpret,
    )(block_expert, num_active, x_padded, w13, w13, down_proj, pw_padded)


@functools.partial(jax.jit, static_argnames=("interpret",))
def _moe(x, expert_indices, expert_weights, w13, down_proj, interpret=False):
    B, S, H = x.shape
    E = w13.shape[0]
    top_k = expert_indices.shape[-1]
    N = B * S
    P = N * top_k
    G = 64
    P_pad = G * _BM

    x_flat = x.reshape(N, H)
    flat_e = expert_indices.reshape(P)
    w_flat = expert_weights.reshape(P)

    order = jnp.argsort(flat_e)
    e_sorted = flat_e[order]
    sizes = jnp.bincount(flat_e, length=E)
    blocks_e = (sizes + _BM - 1) // _BM
    starts_unpad = jnp.cumsum(sizes) - sizes
    block_start_e = jnp.cumsum(blocks_e) - blocks_e
    starts_pad = block_start_e * _BM
    ranks = jnp.arange(P, dtype=jnp.int32) - starts_unpad[e_sorted]
    pos_sorted = (starts_pad[e_sorted] + ranks).astype(jnp.int32)

    tok_padded = jnp.zeros((P_pad,), jnp.int32).at[pos_sorted].set(
        (order // top_k).astype(jnp.int32))
    pw_padded = jnp.zeros((P_pad, 1), jnp.float32).at[pos_sorted, 0].set(
        w_flat[order])
    pos_by_pair = jnp.zeros((P,), jnp.int32).at[order].set(pos_sorted)

    num_active = jnp.sum(blocks_e).astype(jnp.int32).reshape(1)
    block_expert = jnp.minimum(
        jnp.searchsorted(jnp.cumsum(blocks_e), jnp.arange(G), side="right"),
        E - 1).astype(jnp.int32)

    x_padded = x_flat[tok_padded]

    o_padded = _grouped_gemm(x_padded, w13, down_proj, pw_padded,
                             block_expert, num_active, interpret=interpret)

    pos2 = pos_by_pair.reshape(N, top_k)
    out = jnp.sum(o_padded[pos2], axis=1)
    return out.reshape(B, S, H)


def kernel(x, expert_indices, expert_weights, w13, down_proj):
    return _moe(x, expert_indices, expert_weights, w13, down_proj)
